# Initial kernel scaffold; baseline (speedup 1.0000x reference)
import jax
import jax.numpy as jnp
from jax.experimental import pallas as pl


def kernel(z_G, x, proba, uncertainty, degree, edge_index, batch, z_L, k_budget, Wr, br, W1, b1, W2, b2, W3, b3, Wg1, bg1, Wg2, bg2, Wq1, bq1, Wq2, bq2):
    raise NotImplementedError("write your pallas kernel here")


# speedup vs baseline: 1.1126x; 1.1126x over previous
import functools

import jax
import jax.numpy as jnp
from jax import lax
from jax.experimental import pallas as pl

_NB = 512
_DG = 256
_TILE_N = 2048
_TILE_R = 512
_PADC = 128


def _pool_body(z_ref, brow_ref, out_ref):
    i = pl.program_id(0)

    @pl.when(i == 0)
    def _():
        out_ref[...] = jnp.zeros_like(out_ref)

    brow = brow_ref[0]
    gid = lax.broadcasted_iota(jnp.int32, (_NB, _TILE_N), 0).astype(jnp.float32)
    ohT = (gid == brow).astype(jnp.float32)
    out_ref[...] += jax.lax.dot_general(
        ohT, z_ref[...], (((1,), (0,)), ((), ())),
        preferred_element_type=jnp.float32)


def _mlp_body(zr_ref, zl_ref, grow_ref, w1_ref, b1_ref, w2_ref, b2_ref,
              w3_ref, b3_ref, delta_ref):
    i = pl.program_id(0)

    @pl.when(i == 0)
    def _():
        delta_ref[...] = jnp.zeros_like(delta_ref)

    w1a = w1_ref[0:_DG, :]
    w1b = w1_ref[_DG:, :]
    h1 = jax.lax.dot_general(zr_ref[...], w1a, (((1,), (0,)), ((), ())),
                             preferred_element_type=jnp.float32)
    h1 += jax.lax.dot_general(zl_ref[...], w1b, (((1,), (0,)), ((), ())),
                              preferred_element_type=jnp.float32)
    h1 = jnp.maximum(h1 + b1_ref[...], 0.0)
    h2 = jax.lax.dot_general(h1, w2_ref[...], (((1,), (0,)), ((), ())),
                             preferred_element_type=jnp.float32)
    h2 = jnp.maximum(h2 + b2_ref[...], 0.0)
    lg = jax.lax.dot_general(h2, w3_ref[...], (((1,), (0,)), ((), ())),
                             preferred_element_type=jnp.float32)
    lg = lg + b3_ref[...]
    col = lax.broadcasted_iota(jnp.int32, (_TILE_R, _PADC), 1)
    lg = jnp.where(col < 2, lg, -1e30)
    rw = jax.nn.softmax(lg, axis=-1)
    grow = grow_ref[0]
    gid = lax.broadcasted_iota(jnp.int32, (_NB, _TILE_R), 0).astype(jnp.float32)
    ohT = (gid == grow).astype(jnp.float32)
    delta_ref[...] += jax.lax.dot_general(
        ohT, rw, (((1,), (0,)), ((), ())),
        preferred_element_type=jnp.float32)


def _cls_body(esum_ref, dpad_ref, cf_ref,
              wg1_ref, bg1_ref, wg2_ref, bg2_ref,
              wq1_ref, bq1_ref, wq2_ref, bq2_ref,
              lg_ref, lr_ref):
    cf = cf_ref[...]
    emb_g = esum_ref[...] / cf
    hg = jax.lax.dot_general(emb_g, wg1_ref[...], (((1,), (0,)), ((), ())),
                             preferred_element_type=jnp.float32)
    hg = jnp.maximum(hg + bg1_ref[...], 0.0)
    lg_ref[...] = jax.lax.dot_general(hg, wg2_ref[...],
                                      (((1,), (0,)), ((), ())),
                                      preferred_element_type=jnp.float32) + bg2_ref[...]
    emb_r = (esum_ref[...] + dpad_ref[...]) / cf
    hq = jax.lax.dot_general(emb_r, wq1_ref[...], (((1,), (0,)), ((), ())),
                             preferred_element_type=jnp.float32)
    hq = jnp.maximum(hq + bq1_ref[...], 0.0)
    lr_ref[...] = jax.lax.dot_general(hq, wq2_ref[...],
                                      (((1,), (0,)), ((), ())),
                                      preferred_element_type=jnp.float32) + bq2_ref[...]


def kernel(z_G, x, proba, uncertainty, degree, edge_index, batch, z_L,
           k_budget, Wr, br, W1, b1, W2, b2, W3, b3, Wg1, bg1, Wg2, bg2,
           Wq1, bq1, Wq2, bq2):
    n = z_G.shape[0]
    proba_n = proba / jnp.maximum(proba.sum(-1, keepdims=True), 1e-12)
    src, dst = edge_index[0], edge_index[1]
    nb_sum = jnp.zeros((n, proba_n.shape[1]), jnp.float32).at[dst].add(proba_n[src])
    nb_cnt = jnp.zeros((n,), jnp.float32).at[dst].add(1.0)
    nb_cnt = jnp.maximum(nb_cnt, 1.0)[:, None]
    homophily = (proba_n * (nb_sum / nb_cnt)).sum(-1)
    feats = jnp.concatenate([z_G, uncertainty, homophily[:, None],
                             degree[:, None], x], axis=-1)
    scores = jax.nn.sigmoid(feats @ Wr + br)[:, 0]
    sort_key = batch.astype(jnp.float32) * 10.0 + (1.0 - scores)
    order = jnp.argsort(sort_key)
    counts = jnp.bincount(batch, length=_NB)
    starts = jnp.concatenate([jnp.zeros((1,), counts.dtype),
                              jnp.cumsum(counts)[:-1]])
    k_static = z_L.shape[0] // _NB
    offsets = jnp.arange(k_static) + (k_budget - k_static)
    pick = (starts[:, None] + offsets[None, :]).reshape(-1)
    routed_idx = jnp.sort(order[pick])
    routed_mask = jnp.zeros((n,), bool).at[routed_idx].set(True)

    n_pad = ((n + _TILE_N - 1) // _TILE_N) * _TILE_N
    nt = n_pad // _TILE_N
    z_pad = jnp.pad(z_G, ((0, n_pad - n), (0, 0)))
    b_row = jnp.pad(batch.astype(jnp.float32), (0, n_pad - n),
                    constant_values=float(_NB)).reshape(nt, 1, _TILE_N)
    emb_sum = pl.pallas_call(
        _pool_body,
        grid=(nt,),
        in_specs=[pl.BlockSpec((_TILE_N, _DG), lambda i: (i, 0)),
                  pl.BlockSpec((1, 1, _TILE_N), lambda i: (i, 0, 0))],
        out_specs=pl.BlockSpec((_NB, _DG), lambda i: (0, 0)),
        out_shape=jax.ShapeDtypeStruct((_NB, _DG), jnp.float32),
    )(z_pad, b_row)

    r = z_L.shape[0]
    rt = r // _TILE_R
    z_routed = z_G[routed_idx]
    g_row = batch[routed_idx].astype(jnp.float32).reshape(rt, 1, _TILE_R)
    w3p = jnp.pad(W3, ((0, 0), (0, _PADC - W3.shape[1])))
    b3p = jnp.pad(b3, (0, _PADC - b3.shape[0])).reshape(1, _PADC)
    delta = pl.pallas_call(
        _mlp_body,
        grid=(rt,),
        in_specs=[pl.BlockSpec((_TILE_R, _DG), lambda i: (i, 0)),
                  pl.BlockSpec((_TILE_R, z_L.shape[1]), lambda i: (i, 0)),
                  pl.BlockSpec((1, 1, _TILE_R), lambda i: (i, 0, 0)),
                  pl.BlockSpec(W1.shape, lambda i: (0, 0)),
                  pl.BlockSpec((1, b1.shape[0]), lambda i: (0, 0)),
                  pl.BlockSpec(W2.shape, lambda i: (0, 0)),
                  pl.BlockSpec((1, b2.shape[0]), lambda i: (0, 0)),
                  pl.BlockSpec(w3p.shape, lambda i: (0, 0)),
                  pl.BlockSpec((1, _PADC), lambda i: (0, 0))],
        out_specs=pl.BlockSpec((_NB, _PADC), lambda i: (0, 0)),
        out_shape=jax.ShapeDtypeStruct((_NB, _PADC), jnp.float32),
    )(z_routed, z_L, g_row, W1, b1.reshape(1, -1), W2, b2.reshape(1, -1),
      w3p, b3p)

    cf = jnp.maximum(counts.astype(jnp.float32), 1.0)[:, None]
    dpad = jnp.pad(delta, ((0, 0), (0, _DG - _PADC)))
    wg2p = jnp.pad(Wg2, ((0, 0), (0, _PADC - Wg2.shape[1])))
    bg2p = jnp.pad(bg2, (0, _PADC - bg2.shape[0])).reshape(1, _PADC)
    wq2p = jnp.pad(Wq2, ((0, 0), (0, _PADC - Wq2.shape[1])))
    bq2p = jnp.pad(bq2, (0, _PADC - bq2.shape[0])).reshape(1, _PADC)
    full = lambda s: pl.BlockSpec(s, lambda: (0,) * len(s))
    lg, lr = pl.pallas_call(
        _cls_body,
        in_specs=[full((_NB, _DG)), full((_NB, _DG)), full((_NB, 1)),
                  full(Wg1.shape), full((1, bg1.shape[0])),
                  full(wg2p.shape), full((1, _PADC)),
                  full(Wq1.shape), full((1, bq1.shape[0])),
                  full(wq2p.shape), full((1, _PADC))],
        out_specs=[full((_NB, _PADC)), full((_NB, _PADC))],
        out_shape=[jax.ShapeDtypeStruct((_NB, _PADC), jnp.float32),
                   jax.ShapeDtypeStruct((_NB, _PADC), jnp.float32)],
    )(emb_sum, dpad, cf, Wg1, bg1.reshape(1, -1), wg2p, bg2p,
      Wq1, bq1.reshape(1, -1), wq2p, bq2p)

    return lg[:, :2], lr[:, :2], scores, routed_mask


# speedup vs baseline: 7.7744x; 6.9876x over previous
import functools

import jax
import jax.numpy as jnp
from jax import lax
from jax.experimental import pallas as pl
from jax.experimental.pallas import tpu as pltpu
from jax.experimental.pallas import tpu_sc as plsc

_NB = 512
_DG = 256
_TILE_N = 2048
_TILE_R = 512
_PADC = 128


_NPAD = 50176
_NW = 32
_EPW = 25000
_ECH = 5000
_CW = 8
_RPS = _NPAD // 16


def _edge_scatter_body(p16_hbm, src_hbm, dst_hbm, zeros_hbm, out_hbm,
                       sidx_v, rows_v, didx_v, acc):
    cid = lax.axis_index("c")
    sid = lax.axis_index("s")
    zbase = sid * _RPS
    pltpu.sync_copy(zeros_hbm.at[pl.ds(zbase, _RPS)],
                    acc.at[pl.ds(zbase, _RPS)])
    plsc.subcore_barrier()
    ebase = (sid * 2 + cid) * _EPW
    for c in range(_EPW // _ECH):
        b = ebase + c * _ECH
        pltpu.sync_copy(src_hbm.at[pl.ds(b, _ECH)], sidx_v)
        pltpu.sync_copy(p16_hbm.at[sidx_v], rows_v)
        pltpu.sync_copy(dst_hbm.at[pl.ds(b, _ECH)], didx_v)
        pltpu.sync_copy(rows_v, acc.at[didx_v], add=True)
    plsc.subcore_barrier()
    pltpu.sync_copy(acc.at[pl.ds(zbase, _RPS)],
                    out_hbm.at[cid, pl.ds(zbase, _RPS)])


def _edge_scatter(p16, src, dst, zeros):
    mesh = plsc.VectorSubcoreMesh(core_axis_name="c", subcore_axis_name="s")
    k = functools.partial(
        pl.kernel, mesh=mesh,
        compiler_params=pltpu.CompilerParams(use_tc_tiling_on_sc=False),
        out_type=jax.ShapeDtypeStruct((2, _NPAD, _CW), jnp.float32),
        scratch_types=[
            pltpu.VMEM((_ECH,), jnp.int32),
            pltpu.VMEM((_ECH, _CW), jnp.float32),
            pltpu.VMEM((_ECH,), jnp.int32),
            pltpu.VMEM_SHARED((_NPAD, _CW), jnp.float32),
        ],
    )(_edge_scatter_body)
    return k(p16, src, dst, zeros)


def _pool_body(z_ref, brow_ref, out_ref):
    i = pl.program_id(0)

    @pl.when(i == 0)
    def _():
        out_ref[...] = jnp.zeros_like(out_ref)

    brow = brow_ref[0]
    gid = lax.broadcasted_iota(jnp.int32, (_NB, _TILE_N), 0).astype(jnp.float32)
    ohT = (gid == brow).astype(jnp.float32)
    out_ref[...] += jax.lax.dot_general(
        ohT, z_ref[...], (((1,), (0,)), ((), ())),
        preferred_element_type=jnp.float32)


def _mlp_body(zr_ref, zl_ref, grow_ref, w1_ref, b1_ref, w2_ref, b2_ref,
              w3_ref, b3_ref, delta_ref):
    i = pl.program_id(0)

    @pl.when(i == 0)
    def _():
        delta_ref[...] = jnp.zeros_like(delta_ref)

    w1a = w1_ref[0:_DG, :]
    w1b = w1_ref[_DG:, :]
    h1 = jax.lax.dot_general(zr_ref[...], w1a, (((1,), (0,)), ((), ())),
                             preferred_element_type=jnp.float32)
    h1 += jax.lax.dot_general(zl_ref[...], w1b, (((1,), (0,)), ((), ())),
                              preferred_element_type=jnp.float32)
    h1 = jnp.maximum(h1 + b1_ref[...], 0.0)
    h2 = jax.lax.dot_general(h1, w2_ref[...], (((1,), (0,)), ((), ())),
                             preferred_element_type=jnp.float32)
    h2 = jnp.maximum(h2 + b2_ref[...], 0.0)
    lg = jax.lax.dot_general(h2, w3_ref[...], (((1,), (0,)), ((), ())),
                             preferred_element_type=jnp.float32)
    lg = lg + b3_ref[...]
    col = lax.broadcasted_iota(jnp.int32, (_TILE_R, _PADC), 1)
    lg = jnp.where(col < 2, lg, -1e30)
    rw = jax.nn.softmax(lg, axis=-1)
    grow = grow_ref[0]
    gid = lax.broadcasted_iota(jnp.int32, (_NB, _TILE_R), 0).astype(jnp.float32)
    ohT = (gid == grow).astype(jnp.float32)
    delta_ref[...] += jax.lax.dot_general(
        ohT, rw, (((1,), (0,)), ((), ())),
        preferred_element_type=jnp.float32)


def _cls_body(esum_ref, dpad_ref, cf_ref,
              wg1_ref, bg1_ref, wg2_ref, bg2_ref,
              wq1_ref, bq1_ref, wq2_ref, bq2_ref,
              lg_ref, lr_ref):
    cf = cf_ref[...]
    emb_g = esum_ref[...] / cf
    hg = jax.lax.dot_general(emb_g, wg1_ref[...], (((1,), (0,)), ((), ())),
                             preferred_element_type=jnp.float32)
    hg = jnp.maximum(hg + bg1_ref[...], 0.0)
    lg_ref[...] = jax.lax.dot_general(hg, wg2_ref[...],
                                      (((1,), (0,)), ((), ())),
                                      preferred_element_type=jnp.float32) + bg2_ref[...]
    emb_r = (esum_ref[...] + dpad_ref[...]) / cf
    hq = jax.lax.dot_general(emb_r, wq1_ref[...], (((1,), (0,)), ((), ())),
                             preferred_element_type=jnp.float32)
    hq = jnp.maximum(hq + bq1_ref[...], 0.0)
    lr_ref[...] = jax.lax.dot_general(hq, wq2_ref[...],
                                      (((1,), (0,)), ((), ())),
                                      preferred_element_type=jnp.float32) + bq2_ref[...]


def kernel(z_G, x, proba, uncertainty, degree, edge_index, batch, z_L,
           k_budget, Wr, br, W1, b1, W2, b2, W3, b3, Wg1, bg1, Wg2, bg2,
           Wq1, bq1, Wq2, bq2):
    n = z_G.shape[0]
    proba_n = proba / jnp.maximum(proba.sum(-1, keepdims=True), 1e-12)
    src, dst = edge_index[0], edge_index[1]
    p16 = jnp.zeros((_NPAD, _CW), jnp.float32)
    p16 = p16.at[:n, 0:2].set(proba_n).at[:n, 2].set(1.0)
    parts = _edge_scatter(p16, src, dst, jnp.zeros((_NPAD, _CW), jnp.float32))
    nb = parts[0] + parts[1]
    nb_sum = nb[:n, 0:2]
    nb_cnt = jnp.maximum(nb[:n, 2], 1.0)[:, None]
    homophily = (proba_n * (nb_sum / nb_cnt)).sum(-1)
    feats = jnp.concatenate([z_G, uncertainty, homophily[:, None],
                             degree[:, None], x], axis=-1)
    scores = jax.nn.sigmoid(feats @ Wr + br)[:, 0]
    sort_key = batch.astype(jnp.float32) * 10.0 + (1.0 - scores)
    order = jnp.argsort(sort_key)
    counts = jnp.bincount(batch, length=_NB)
    starts = jnp.concatenate([jnp.zeros((1,), counts.dtype),
                              jnp.cumsum(counts)[:-1]])
    k_static = z_L.shape[0] // _NB
    offsets = jnp.arange(k_static) + (k_budget - k_static)
    pick = (starts[:, None] + offsets[None, :]).reshape(-1)
    routed_idx = jnp.sort(order[pick])
    routed_mask = jnp.zeros((n,), bool).at[routed_idx].set(True)

    n_pad = ((n + _TILE_N - 1) // _TILE_N) * _TILE_N
    nt = n_pad // _TILE_N
    z_pad = jnp.pad(z_G, ((0, n_pad - n), (0, 0)))
    b_row = jnp.pad(batch.astype(jnp.float32), (0, n_pad - n),
                    constant_values=float(_NB)).reshape(nt, 1, _TILE_N)
    emb_sum = pl.pallas_call(
        _pool_body,
        grid=(nt,),
        in_specs=[pl.BlockSpec((_TILE_N, _DG), lambda i: (i, 0)),
                  pl.BlockSpec((1, 1, _TILE_N), lambda i: (i, 0, 0))],
        out_specs=pl.BlockSpec((_NB, _DG), lambda i: (0, 0)),
        out_shape=jax.ShapeDtypeStruct((_NB, _DG), jnp.float32),
    )(z_pad, b_row)

    r = z_L.shape[0]
    rt = r // _TILE_R
    z_routed = z_G[routed_idx]
    g_row = batch[routed_idx].astype(jnp.float32).reshape(rt, 1, _TILE_R)
    w3p = jnp.pad(W3, ((0, 0), (0, _PADC - W3.shape[1])))
    b3p = jnp.pad(b3, (0, _PADC - b3.shape[0])).reshape(1, _PADC)
    delta = pl.pallas_call(
        _mlp_body,
        grid=(rt,),
        in_specs=[pl.BlockSpec((_TILE_R, _DG), lambda i: (i, 0)),
                  pl.BlockSpec((_TILE_R, z_L.shape[1]), lambda i: (i, 0)),
                  pl.BlockSpec((1, 1, _TILE_R), lambda i: (i, 0, 0)),
                  pl.BlockSpec(W1.shape, lambda i: (0, 0)),
                  pl.BlockSpec((1, b1.shape[0]), lambda i: (0, 0)),
                  pl.BlockSpec(W2.shape, lambda i: (0, 0)),
                  pl.BlockSpec((1, b2.shape[0]), lambda i: (0, 0)),
                  pl.BlockSpec(w3p.shape, lambda i: (0, 0)),
                  pl.BlockSpec((1, _PADC), lambda i: (0, 0))],
        out_specs=pl.BlockSpec((_NB, _PADC), lambda i: (0, 0)),
        out_shape=jax.ShapeDtypeStruct((_NB, _PADC), jnp.float32),
    )(z_routed, z_L, g_row, W1, b1.reshape(1, -1), W2, b2.reshape(1, -1),
      w3p, b3p)

    cf = jnp.maximum(counts.astype(jnp.float32), 1.0)[:, None]
    dpad = jnp.pad(delta, ((0, 0), (0, _DG - _PADC)))
    wg2p = jnp.pad(Wg2, ((0, 0), (0, _PADC - Wg2.shape[1])))
    bg2p = jnp.pad(bg2, (0, _PADC - bg2.shape[0])).reshape(1, _PADC)
    wq2p = jnp.pad(Wq2, ((0, 0), (0, _PADC - Wq2.shape[1])))
    bq2p = jnp.pad(bq2, (0, _PADC - bq2.shape[0])).reshape(1, _PADC)
    full = lambda s: pl.BlockSpec(s, lambda: (0,) * len(s))
    lg, lr = pl.pallas_call(
        _cls_body,
        in_specs=[full((_NB, _DG)), full((_NB, _DG)), full((_NB, 1)),
                  full(Wg1.shape), full((1, bg1.shape[0])),
                  full(wg2p.shape), full((1, _PADC)),
                  full(Wq1.shape), full((1, bq1.shape[0])),
                  full(wq2p.shape), full((1, _PADC))],
        out_specs=[full((_NB, _PADC)), full((_NB, _PADC))],
        out_shape=[jax.ShapeDtypeStruct((_NB, _PADC), jnp.float32),
                   jax.ShapeDtypeStruct((_NB, _PADC), jnp.float32)],
    )(emb_sum, dpad, cf, Wg1, bg1.reshape(1, -1), wg2p, bg2p,
      Wq1, bq1.reshape(1, -1), wq2p, bq2p)

    return lg[:, :2], lr[:, :2], scores, routed_mask
